# baseline (speedup 1.0000x reference)
import jax
import jax.numpy as jnp
from jax import lax
from jax.experimental import pallas as pl
from jax.experimental.pallas import tpu as pltpu
from jax.experimental.pallas import tpu_sc as plsc

B = 16384
EMBED = 16
H = 64
VOCAB = 1000000
LANES_PAD = 1000064

_info = plsc.get_sparse_core_info()
_NC, _NS = _info.num_cores, _info.num_subcores
_NW = _NC * _NS
_CHUNK = 1024
_NCH = 977
_NCPW = 31
_LAST_OFF = 999040
_CAP = 768
_NSCAT = _CAP // 128
_OUTROWS = B + 128
_IPIECE = 4096


def _gather_body(idx_hbm, tableT_hbm, out_hbm, ibuf, midx, mposl, mpos2,
                 chunk_v, stage_v, sem):
    wid = lax.axis_index("s") * _NC + lax.axis_index("c")
    start_c = (wid * _NCH) >> 5
    end_c = ((wid + 1) * _NCH) >> 5
    lo = start_c * _CHUNK
    hi = end_c * _CHUNK
    lanes = lax.iota(jnp.int32, 16)

    def prefill(g, carry):
        dummy = B + ((g * 16 + lanes) & 127)
        mposl[pl.ds(g * 16, 16)] = dummy
        midx[pl.ds(g * 16, 16)] = jnp.full((16,), 0, jnp.int32) + lo
        return carry

    lax.fori_loop(0, _CAP // 16, prefill, 0)

    def scan_piece(p, off):
        pltpu.sync_copy(idx_hbm.at[pl.ds(p * _IPIECE, _IPIECE)], ibuf)

        def scan_group(g, off):
            v = ibuf[pl.ds(g * 16, 16)]
            m = jnp.logical_and(v >= lo, v < hi)
            woff = jnp.minimum(off, _CAP - 16)
            plsc.store_compressed(midx.at[pl.ds(woff, 16)], v, mask=m)
            pos = p * _IPIECE + g * 16 + lanes
            plsc.store_compressed(mposl.at[pl.ds(woff, 16)], pos, mask=m)
            return off + plsc.all_reduce_population_count(m)[0]

        return lax.fori_loop(0, _IPIECE // 16, scan_group, off)

    lax.fori_loop(0, B // _IPIECE, scan_piece, jnp.int32(0))

    def pos2d(g, carry):
        r = g // 8
        l = g % 8
        mpos2[r, pl.ds(l * 16, 16)] = mposl[pl.ds(g * 16, 16)]
        return carry

    lax.fori_loop(0, _CAP // 16, pos2d, 0)

    def chunk_step(ci, carry):
        c = jnp.minimum(start_c + ci, end_c - 1)
        c_lo = c * _CHUNK
        o_c = pl.multiple_of(jnp.minimum(c_lo, _LAST_OFF), 128)
        pltpu.sync_copy(tableT_hbm.at[:, pl.ds(o_c, _CHUNK)], chunk_v)

        def group_step(g, carry):
            mv = midx[pl.ds(g * 16, 16)]
            inm = jnp.logical_and(mv >= c_lo, mv < c_lo + _CHUNK)
            cnt = plsc.all_reduce_population_count(inm)[0]

            @pl.when(cnt > 0)
            def _():
                loc = mv - o_c
                rows = g * 16 + lanes
                for f in range(EMBED):
                    vals = plsc.load_gather(
                        chunk_v, [jnp.full((16,), f, jnp.int32), loc],
                        mask=inm,
                    )
                    plsc.store_scatter(
                        stage_v, [rows, jnp.full((16,), f, jnp.int32)],
                        vals, mask=inm,
                    )

            return carry

        lax.fori_loop(0, _CAP // 16, group_step, 0)
        return carry

    lax.fori_loop(0, _NCPW, chunk_step, 0)

    copies = []
    for s in range(_NSCAT):
        copies.append(
            pltpu.async_copy(
                stage_v.at[pl.ds(s * 128, 128)],
                out_hbm.at[mpos2.at[s]],
                sem,
            )
        )
    for cp in copies:
        cp.wait()


_sc_gather = pl.kernel(
    _gather_body,
    mesh=plsc.VectorSubcoreMesh(core_axis_name="c", subcore_axis_name="s"),
    out_type=jax.ShapeDtypeStruct((_OUTROWS, 128), jnp.float32),
    scratch_types=[
        pltpu.VMEM((_IPIECE,), jnp.int32),
        pltpu.VMEM((_CAP,), jnp.int32),
        pltpu.VMEM((_CAP,), jnp.int32),
        pltpu.VMEM((_NSCAT, 128), jnp.int32),
        pltpu.VMEM((EMBED, _CHUNK), jnp.float32),
        pltpu.VMEM((_CAP, 128), jnp.float32),
        pltpu.SemaphoreType.DMA,
    ],
    compiler_params=pltpu.CompilerParams(needs_layout_passes=False),
)

_BLK = 4096


def _mlp_body(e_ref, W1_ref, b1_ref, W2_ref, b2_ref, out_ref):
    e = e_ref[...][:, :EMBED]
    h = jnp.dot(e, W1_ref[...], preferred_element_type=jnp.float32)
    h = jnp.maximum(h + b1_ref[...], 0.0)
    o = jnp.dot(h, W2_ref[...], preferred_element_type=jnp.float32)
    out_ref[...] = o + b2_ref[...]


_tc_mlp = pl.pallas_call(
    _mlp_body,
    grid=(B // _BLK,),
    in_specs=[
        pl.BlockSpec((_BLK, 128), lambda i: (i, 0)),
        pl.BlockSpec((EMBED, H), lambda i: (0, 0)),
        pl.BlockSpec((1, H), lambda i: (0, 0)),
        pl.BlockSpec((H, 1), lambda i: (0, 0)),
        pl.BlockSpec((1, 1), lambda i: (0, 0)),
    ],
    out_specs=pl.BlockSpec((_BLK, 1), lambda i: (i, 0)),
    out_shape=jax.ShapeDtypeStruct((B, 1), jnp.float32),
)


@jax.jit
def kernel(x, table, W1, b1, W2, b2):
    idx = x.astype(jnp.int32).reshape(B)
    e128 = _sc_gather(idx, table.T)
    o = _tc_mlp(e128, W1, b1.reshape(1, H), W2, b2.reshape(1, 1))
    return o.reshape(B)


# speedup vs baseline: 1.1750x; 1.1750x over previous
import jax
import jax.numpy as jnp
from jax import lax
from jax.experimental import pallas as pl
from jax.experimental.pallas import tpu as pltpu
from jax.experimental.pallas import tpu_sc as plsc

B = 16384
EMBED = 16
H = 64

_info = plsc.get_sparse_core_info()
_NC, _NS = _info.num_cores, _info.num_subcores
_NW = _NC * _NS
_BPW = B // _NW
_GRP = 16


def _gather_body(idx_hbm, tableT_hbm, w1_hbm, b1_hbm, w2_hbm, b2_hbm,
                 out_hbm, idx_v, o_v, w1_v, b1_v, w2_v, b2_v, sem, *slots):
    wid = lax.axis_index("s") * _NC + lax.axis_index("c")
    base = wid * _BPW
    pltpu.sync_copy(idx_hbm.at[pl.ds(base, _BPW)], idx_v)
    pltpu.sync_copy(w1_hbm, w1_v)
    pltpu.sync_copy(b1_hbm, b1_v)
    pltpu.sync_copy(w2_hbm, w2_v)
    pltpu.sync_copy(b2_hbm, b2_v)
    lanes = lax.iota(jnp.int32, 16)

    def group_step(g, carry):
        p0 = g * _GRP
        v = idx_v[pl.ds(p0, _GRP)]
        copies = []
        for j in range(_GRP):
            k = v[j]
            lane_base = pl.multiple_of(
                lax.shift_left(lax.shift_right_logical(k, 7), 7), 128
            )
            copies.append(
                pltpu.async_copy(
                    tableT_hbm.at[:, pl.ds(lane_base, 128)], slots[j], sem
                )
            )
        for j in range(_GRP):
            copies[j].wait()
            col = jnp.bitwise_and(v[j], 127)
            vals = plsc.load_gather(
                slots[j], [lanes, jnp.full((16,), 0, jnp.int32) + col]
            )
            acc = jnp.full((16,), 0.0, jnp.float32)
            for m in range(H // 16):
                hm = b1_v[pl.ds(m * 16, 16)]
                for k in range(EMBED):
                    hm = hm + vals[k] * w1_v[k, pl.ds(m * 16, 16)]
                hm = jnp.maximum(hm, 0.0)
                acc = acc + hm * w2_v[pl.ds(m * 16, 16)]
            o = jnp.sum(acc)
            plsc.store_scatter(
                o_v, [jnp.full((16,), p0 + j, jnp.int32)],
                jnp.full((16,), 0.0, jnp.float32) + o,
                mask=lanes == 0,
            )
        return carry

    lax.fori_loop(0, _BPW // _GRP, group_step, 0)

    def bias_step(g, carry):
        o_v[pl.ds(g * 16, 16)] = o_v[pl.ds(g * 16, 16)] + b2_v[...]
        return carry

    lax.fori_loop(0, _BPW // 16, bias_step, 0)
    pltpu.sync_copy(o_v, out_hbm.at[pl.ds(base, _BPW)])


_sc_gather_mlp = pl.kernel(
    _gather_body,
    mesh=plsc.VectorSubcoreMesh(core_axis_name="c", subcore_axis_name="s"),
    out_type=jax.ShapeDtypeStruct((B,), jnp.float32),
    scratch_types=[
        pltpu.VMEM((_BPW,), jnp.int32),
        pltpu.VMEM((_BPW,), jnp.float32),
        pltpu.VMEM((EMBED, H), jnp.float32),
        pltpu.VMEM((H,), jnp.float32),
        pltpu.VMEM((H,), jnp.float32),
        pltpu.VMEM((16,), jnp.float32),
        pltpu.SemaphoreType.DMA,
    ] + [pltpu.VMEM((EMBED, 128), jnp.float32) for _ in range(_GRP)],
    compiler_params=pltpu.CompilerParams(needs_layout_passes=False),
)


@jax.jit
def kernel(x, table, W1, b1, W2, b2):
    idx = x.astype(jnp.int32).reshape(B)
    return _sc_gather_mlp(
        idx, table.T, W1, b1, W2.reshape(H),
        jnp.broadcast_to(b2, (16,)),
    )


# speedup vs baseline: 1.4582x; 1.2410x over previous
import jax
import jax.numpy as jnp
from jax import lax
from jax.experimental import pallas as pl
from jax.experimental.pallas import tpu as pltpu
from jax.experimental.pallas import tpu_sc as plsc

B = 16384
EMBED = 16
H = 64

_info = plsc.get_sparse_core_info()
_NC, _NS = _info.num_cores, _info.num_subcores
_NW = _NC * _NS
_BPW = B // _NW
_GRP = 16


def _gather_body(idx_hbm, tableT_hbm, out_hbm, idx_v, e_v, sem, *slots):
    wid = lax.axis_index("s") * _NC + lax.axis_index("c")
    base = wid * _BPW
    pltpu.sync_copy(idx_hbm.at[pl.ds(base, _BPW)], idx_v)
    lanes = lax.iota(jnp.int32, 16)

    def group_step(g, carry):
        p0 = g * _GRP
        v = idx_v[pl.ds(p0, _GRP)]
        copies = []
        for j in range(_GRP):
            k = v[j]
            lane_base = pl.multiple_of(
                lax.shift_left(lax.shift_right_logical(k, 7), 7), 128
            )
            copies.append(
                pltpu.async_copy(
                    tableT_hbm.at[:, pl.ds(lane_base, 128)], slots[j], sem
                )
            )
        for j in range(_GRP):
            copies[j].wait()
            col = jnp.bitwise_and(v[j], 127)
            vals = plsc.load_gather(
                slots[j], [lanes, jnp.full((16,), 0, jnp.int32) + col]
            )
            e_v[p0 + j, :] = vals
        return carry

    lax.fori_loop(0, _BPW // _GRP, group_step, 0)
    pltpu.sync_copy(e_v, out_hbm.at[pl.ds(base, _BPW), :])


_sc_gather = pl.kernel(
    _gather_body,
    mesh=plsc.VectorSubcoreMesh(core_axis_name="c", subcore_axis_name="s"),
    out_type=jax.ShapeDtypeStruct((B, EMBED), jnp.float32),
    scratch_types=[
        pltpu.VMEM((_BPW,), jnp.int32),
        pltpu.VMEM((_BPW, EMBED), jnp.float32),
        pltpu.SemaphoreType.DMA,
    ] + [pltpu.VMEM((EMBED, 128), jnp.float32) for _ in range(_GRP)],
    compiler_params=pltpu.CompilerParams(needs_layout_passes=False),
)

_BLK = 4096


def _mlp_body(e_ref, W1_ref, b1_ref, W2_ref, b2_ref, out_ref):
    h = jnp.dot(e_ref[...], W1_ref[...], preferred_element_type=jnp.float32)
    h = jnp.maximum(h + b1_ref[...], 0.0)
    o = jnp.dot(h, W2_ref[...], preferred_element_type=jnp.float32)
    out_ref[...] = o + b2_ref[...]


_tc_mlp = pl.pallas_call(
    _mlp_body,
    grid=(B // _BLK,),
    in_specs=[
        pl.BlockSpec((_BLK, EMBED), lambda i: (i, 0)),
        pl.BlockSpec((EMBED, H), lambda i: (0, 0)),
        pl.BlockSpec((1, H), lambda i: (0, 0)),
        pl.BlockSpec((H, 1), lambda i: (0, 0)),
        pl.BlockSpec((1, 1), lambda i: (0, 0)),
    ],
    out_specs=pl.BlockSpec((_BLK, 1), lambda i: (i, 0)),
    out_shape=jax.ShapeDtypeStruct((B, 1), jnp.float32),
)


@jax.jit
def kernel(x, table, W1, b1, W2, b2):
    idx = x.astype(jnp.int32).reshape(B)
    e = _sc_gather(idx, table.T)
    o = _tc_mlp(e, W1, b1.reshape(1, H), W2, b2.reshape(1, 1))
    return o.reshape(B)
